# baseline (speedup 1.0000x reference)
import functools
import math

import jax
import jax.numpy as jnp
from jax import lax
from jax.experimental import pallas as pl
from jax.experimental.pallas import tpu as pltpu
from jax.experimental.pallas import tpu_sc as plsc

N = 10000
D = 128
E = 320000
B = 16384
LAYERS = 2
DECAY = math.exp(-1.0)

NC = 2
NS = 16
NW = NC * NS
K = 128
CPT = -(-E // (NW * K))
E_PAD = CPT * NW * K
KC = 32
CPT2 = 315
E_PAD2 = CPT2 * NW * KC
N_PAD = 10112
ROWS_PER_TILE = N_PAD // NS

_mesh = plsc.VectorSubcoreMesh(core_axis_name="c", subcore_axis_name="s")


@functools.partial(
    pl.kernel,
    mesh=_mesh,
    out_type=jax.ShapeDtypeStruct((NC, N_PAD, D), jnp.float32),
    compiler_params=pltpu.CompilerParams(use_tc_tiling_on_sc=False),
    scratch_types=[
        pltpu.VMEM_SHARED((N_PAD, D), jnp.float32),
        pltpu.VMEM_SHARED((N_PAD, D // 2), jnp.int32),
        pltpu.VMEM((4, KC), jnp.int32),
        pltpu.VMEM((4, KC), jnp.int32),
        pltpu.VMEM((4, KC), jnp.float32),
        pltpu.VMEM((KC, D // 2), jnp.int32),
        pltpu.VMEM((KC, D // 2), jnp.int32),
        pltpu.VMEM((KC, D), jnp.float32),
        pltpu.SemaphoreType.DMA,
        pltpu.SemaphoreType.DMA,
        pltpu.SemaphoreType.DMA,
        pltpu.SemaphoreType.DMA,
        pltpu.SemaphoreType.DMA,
        pltpu.SemaphoreType.DMA,
    ],
)
def _sc_layer(h_hbm, src_hbm, dst_hbm, w_hbm, zero_hbm, out_hbm,
              acc, tab, srcs, dsts, ws, rows0, rows1, rowsf,
              semi0, semi1, semi2, semi3, semr0, semr1):
    c = lax.axis_index("c")
    s = lax.axis_index("s")
    wid = c * NS + s

    r0 = pl.multiple_of(s * ROWS_PER_TILE, 8)
    pltpu.sync_copy(zero_hbm.at[pl.ds(r0, ROWS_PER_TILE)],
                    acc.at[pl.ds(r0, ROWS_PER_TILE)])
    pltpu.sync_copy(h_hbm.at[pl.ds(r0, ROWS_PER_TILE)],
                    tab.at[pl.ds(r0, ROWS_PER_TILE)])
    plsc.subcore_barrier()

    semis = (semi0, semi1, semi2, semi3)
    rows_bufs = (rows0, rows1)
    semrs = (semr0, semr1)

    def load_idx(i, b):
        base = pl.multiple_of((wid * CPT2 + i) * KC, 8)
        pltpu.async_copy(src_hbm.at[pl.ds(base, KC)], srcs.at[b], semis[b])
        pltpu.async_copy(dst_hbm.at[pl.ds(base, KC)], dsts.at[b], semis[b])
        pltpu.async_copy(w_hbm.at[pl.ds(base, KC)], ws.at[b], semis[b])

    def wait_idx(i, b):
        base = pl.multiple_of((wid * CPT2 + i) * KC, 8)
        pltpu.make_async_copy(src_hbm.at[pl.ds(base, KC)], srcs.at[b],
                              semis[b]).wait()
        pltpu.make_async_copy(dst_hbm.at[pl.ds(base, KC)], dsts.at[b],
                              semis[b]).wait()
        pltpu.make_async_copy(w_hbm.at[pl.ds(base, KC)], ws.at[b],
                              semis[b]).wait()

    mask_hi = jnp.int32(-65536)

    def expand_scale(rows, wv):
        for g in range(KC // 16):
            w16 = wv[pl.ds(g * 16, 16)]
            for l in range(16):
                e = g * 16 + l
                web = jnp.full((16,), w16[l], jnp.float32)
                for j in range(D // 32):
                    x = rows[e, pl.ds(j * 16, 16)]
                    lo = lax.bitcast_convert_type(x << 16, jnp.float32)
                    hi = lax.bitcast_convert_type(x & mask_hi, jnp.float32)
                    rowsf[e, pl.ds(j * 32, 16)] = lo * web
                    rowsf[e, pl.ds(j * 32 + 16, 16)] = hi * web

    def gather_rows(b, r):
        pltpu.async_copy(tab.at[srcs.at[b]], rows_bufs[r], semrs[r])

    def wait_rows(b, r):
        pltpu.make_async_copy(tab.at[srcs.at[b]], rows_bufs[r],
                              semrs[r]).wait()

    def block(i, bi):
        wait_idx(i + 1, (bi + 1) % 4)
        gather_rows((bi + 1) % 4, (bi + 1) % 2)
        wait_rows(bi, bi % 2)
        expand_scale(rows_bufs[bi % 2], ws.at[bi])
        pltpu.sync_copy(rowsf, acc.at[dsts.at[bi]], add=True)
        load_idx(i + 3, (bi + 3) % 4)

    load_idx(0, 0)
    load_idx(1, 1)
    load_idx(2, 2)
    wait_idx(0, 0)
    gather_rows(0, 0)

    def quad_body(p, carry):
        i = p * 4
        block(i, 0)
        block(i + 1, 1)
        block(i + 2, 2)
        block(i + 3, 3)
        return carry

    lax.fori_loop(0, (CPT2 - 3) // 4, quad_body, 0)
    e = CPT2 - 3
    for q in range(3):
        i = e + q
        bi = i % 4
        if q < 2:
            wait_idx(i + 1, (bi + 1) % 4)
            gather_rows((bi + 1) % 4, (bi + 1) % 2)
        wait_rows(bi, bi % 2)
        expand_scale(rows_bufs[bi % 2], ws.at[bi])
        pltpu.sync_copy(rowsf, acc.at[dsts.at[bi]], add=True)

    plsc.subcore_barrier()
    pltpu.sync_copy(acc.at[pl.ds(r0, ROWS_PER_TILE)],
                    out_hbm.at[c, pl.ds(r0, ROWS_PER_TILE)])


G = 2 * B
G_PER_TILE = G // NW
G_CHUNKS = G_PER_TILE // K


@functools.partial(
    pl.kernel,
    mesh=_mesh,
    out_type=jax.ShapeDtypeStruct((G, D), jnp.float32),
    scratch_types=[
        pltpu.VMEM((K,), jnp.int32),
        pltpu.VMEM((K, D), jnp.float32),
        pltpu.SemaphoreType.DMA,
    ],
)
def _sc_gather(tab_hbm, idx_hbm, out_hbm, idxv, rows, sem):
    c = lax.axis_index("c")
    s = lax.axis_index("s")
    wid = c * NS + s

    def body(i, carry):
        base = pl.multiple_of(wid * G_PER_TILE + i * K, 8)
        pltpu.sync_copy(idx_hbm.at[pl.ds(base, K)], idxv)
        pltpu.async_copy(tab_hbm.at[idxv], rows, sem).wait()
        pltpu.sync_copy(rows, out_hbm.at[pl.ds(base, K)])
        return carry

    lax.fori_loop(0, G_CHUNKS, body, 0)


_CB = 1264


def _pack_cols(v):
    a = jnp.concatenate([v[:, 32 * j:32 * j + 16] for j in range(D // 32)],
                        axis=1)
    b = jnp.concatenate(
        [v[:, 32 * j + 16:32 * j + 32] for j in range(D // 32)], axis=1)
    au = lax.bitcast_convert_type(a.astype(jnp.bfloat16),
                                  jnp.uint16).astype(jnp.uint32)
    bu = lax.bitcast_convert_type(b.astype(jnp.bfloat16),
                                  jnp.uint16).astype(jnp.uint32)
    return lax.bitcast_convert_type(au | (bu << 16), jnp.int32)


def _combine_body(p0_ref, p1_ref, h_ref, o_ref, o32_ref):
    v = DECAY * (p0_ref[...] + p1_ref[...]) + h_ref[...]
    o_ref[...] = v
    o32_ref[...] = _pack_cols(v)


def _combine(p, h):
    spec = pl.BlockSpec((_CB, D), lambda i: (i, 0))
    return pl.pallas_call(
        _combine_body,
        out_shape=[
            jax.ShapeDtypeStruct((N_PAD, D), jnp.float32),
            jax.ShapeDtypeStruct((N_PAD, D // 2), jnp.int32),
        ],
        grid=(N_PAD // _CB,),
        in_specs=[spec, spec, spec],
        out_specs=[spec, pl.BlockSpec((_CB, D // 2), lambda i: (i, 0))],
    )(p[0], p[1], h)


_MB = 1024


def _mlp_body(xs_ref, xd_ref, w1a_ref, w1b_ref, b1_ref, w2_ref, b2_ref,
              lab_ref, o_ref):
    i = pl.program_id(0)
    h = jnp.dot(xs_ref[...], w1a_ref[...], preferred_element_type=jnp.float32)
    h = h + jnp.dot(xd_ref[...], w1b_ref[...],
                    preferred_element_type=jnp.float32)
    h = jnp.maximum(h + b1_ref[...], 0.0)
    l0 = jnp.sum(h * w2_ref[0:1, :], axis=1) + b2_ref[0, 0]
    l1 = jnp.sum(h * w2_ref[1:2, :], axis=1) + b2_ref[0, 1]
    m = jnp.maximum(l0, l1)
    e0 = jnp.exp(l0 - m)
    e1 = jnp.exp(l1 - m)
    inv = 1.0 / (e0 + e1)
    p0 = e0 * inv
    p1 = e1 * inv
    m2 = jnp.maximum(p0, p1)
    ls = m2 + jnp.log(jnp.exp(p0 - m2) + jnp.exp(p1 - m2))
    lab = lab_ref[...][:, 0]
    lp = jnp.where(lab == 0, p0, p1) - ls
    part = (-jnp.sum(lp) * (1.0 / B)).reshape(1, 1)

    @pl.when(i == 0)
    def _init():
        o_ref[...] = jnp.zeros((1, 1), jnp.float32)

    o_ref[...] += part


def _mlp_loss(xsrc, xdst, w1a, w1b, b1, w2t, b2, labels):
    bspec = pl.BlockSpec((_MB, D), lambda i: (i, 0))
    wspec = pl.BlockSpec((D, D), lambda i: (0, 0))
    return pl.pallas_call(
        _mlp_body,
        out_shape=jax.ShapeDtypeStruct((1, 1), jnp.float32),
        grid=(B // _MB,),
        in_specs=[
            bspec, bspec, wspec, wspec,
            pl.BlockSpec((1, D), lambda i: (0, 0)),
            pl.BlockSpec((2, D), lambda i: (0, 0)),
            pl.BlockSpec((1, 2), lambda i: (0, 0)),
            pl.BlockSpec((_MB, 1), lambda i: (i, 0)),
        ],
        out_specs=pl.BlockSpec((1, 1), lambda i: (0, 0)),
    )(xsrc, xdst, w1a, w1b, b1, w2t, b2, labels)


def kernel(pairs, labels, edge_index, edge_weight, embedding, W1, b1, W2, b2):
    src = edge_index[0].astype(jnp.int32)
    dst = edge_index[1].astype(jnp.int32)
    w = edge_weight.astype(jnp.float32)

    pad = E_PAD2 - E
    src = jnp.concatenate([src, jnp.zeros((pad,), jnp.int32)])
    dst = jnp.concatenate([dst, jnp.zeros((pad,), jnp.int32)])
    w = jnp.concatenate([w, jnp.zeros((pad,), jnp.float32)])
    zeros = jnp.zeros((N_PAD, D), jnp.float32)

    h = jnp.concatenate([embedding, jnp.zeros((N_PAD - N, D), jnp.float32)])
    h32 = _pack_cols(h)
    for _ in range(LAYERS):
        p = _sc_layer(h32, src, dst, w, zeros)
        h, h32 = _combine(p, h)

    idx_all = jnp.concatenate(
        [pairs[:, 0].astype(jnp.int32), pairs[:, 1].astype(jnp.int32)])
    x = _sc_gather(h, idx_all)
    xsrc = x[:B]
    xdst = x[B:]

    w1a = W1[:D]
    w1b = W1[D:]
    b1r = b1.reshape(1, D)
    w2t = W2.T
    b2r = b2.reshape(1, 2)
    lab = labels.astype(jnp.int32).reshape(B, 1)

    loss = _mlp_loss(xsrc, xdst, w1a, w1b, b1r, w2t, b2r, lab)
    return loss[0, 0]


# speedup vs baseline: 1.0324x; 1.0324x over previous
import functools
import math

import jax
import jax.numpy as jnp
from jax import lax
from jax.experimental import pallas as pl
from jax.experimental.pallas import tpu as pltpu
from jax.experimental.pallas import tpu_sc as plsc

N = 10000
D = 128
E = 320000
B = 16384
LAYERS = 2
DECAY = math.exp(-1.0)

NC = 2
NS = 16
NW = NC * NS
K = 128
CPT = -(-E // (NW * K))
E_PAD = CPT * NW * K
KC = 32
CPT2 = 315
E_PAD2 = CPT2 * NW * KC
N_PAD = 10112
ROWS_PER_TILE = N_PAD // NS

_mesh = plsc.VectorSubcoreMesh(core_axis_name="c", subcore_axis_name="s")


@functools.partial(
    pl.kernel,
    mesh=_mesh,
    out_type=jax.ShapeDtypeStruct((NC, N_PAD, D), jnp.float32),
    compiler_params=pltpu.CompilerParams(use_tc_tiling_on_sc=False),
    scratch_types=[
        pltpu.VMEM_SHARED((N_PAD, D), jnp.float32),
        pltpu.VMEM_SHARED((N_PAD, D // 2), jnp.int32),
        pltpu.VMEM((4, KC), jnp.int32),
        pltpu.VMEM((4, KC), jnp.int32),
        pltpu.VMEM((4, KC), jnp.float32),
        pltpu.VMEM((KC, D // 2), jnp.int32),
        pltpu.VMEM((KC, D // 2), jnp.int32),
        pltpu.VMEM((KC, D), jnp.float32),
        pltpu.SemaphoreType.DMA,
        pltpu.SemaphoreType.DMA,
        pltpu.SemaphoreType.DMA,
        pltpu.SemaphoreType.DMA,
        pltpu.SemaphoreType.DMA,
        pltpu.SemaphoreType.DMA,
    ],
)
def _sc_layer(h_hbm, src_hbm, dst_hbm, w_hbm, zero_hbm, out_hbm,
              acc, tab, srcs, dsts, ws, rows0, rows1, rowsf,
              semi0, semi1, semi2, semi3, semr0, semr1):
    c = lax.axis_index("c")
    s = lax.axis_index("s")
    wid = c * NS + s

    r0 = pl.multiple_of(s * ROWS_PER_TILE, 8)
    pltpu.sync_copy(zero_hbm.at[pl.ds(r0, ROWS_PER_TILE)],
                    acc.at[pl.ds(r0, ROWS_PER_TILE)])
    pltpu.sync_copy(h_hbm.at[pl.ds(r0, ROWS_PER_TILE)],
                    tab.at[pl.ds(r0, ROWS_PER_TILE)])
    plsc.subcore_barrier()

    semis = (semi0, semi1, semi2, semi3)
    rows_bufs = (rows0, rows1)
    semrs = (semr0, semr1)

    def load_idx(i, b):
        base = pl.multiple_of((wid * CPT2 + i) * KC, 8)
        pltpu.async_copy(src_hbm.at[pl.ds(base, KC)], srcs.at[b], semis[b])
        pltpu.async_copy(dst_hbm.at[pl.ds(base, KC)], dsts.at[b], semis[b])
        pltpu.async_copy(w_hbm.at[pl.ds(base, KC)], ws.at[b], semis[b])

    def wait_idx(i, b):
        base = pl.multiple_of((wid * CPT2 + i) * KC, 8)
        pltpu.make_async_copy(src_hbm.at[pl.ds(base, KC)], srcs.at[b],
                              semis[b]).wait()
        pltpu.make_async_copy(dst_hbm.at[pl.ds(base, KC)], dsts.at[b],
                              semis[b]).wait()
        pltpu.make_async_copy(w_hbm.at[pl.ds(base, KC)], ws.at[b],
                              semis[b]).wait()

    mask_hi = jnp.int32(-65536)

    def expand_scale(rows, wv):
        for g in range(KC // 16):
            w16 = wv[pl.ds(g * 16, 16)]
            for l in range(16):
                e = g * 16 + l
                web = jnp.full((16,), w16[l], jnp.float32)
                for j in range(D // 32):
                    x = rows[e, pl.ds(j * 16, 16)]
                    lo = lax.bitcast_convert_type(x << 16, jnp.float32)
                    hi = lax.bitcast_convert_type(x & mask_hi, jnp.float32)
                    rowsf[e, pl.ds(j * 32, 16)] = lo * web
                    rowsf[e, pl.ds(j * 32 + 16, 16)] = hi * web

    def gather_rows(b, r):
        pltpu.async_copy(tab.at[srcs.at[b]], rows_bufs[r], semrs[r])

    def wait_rows(b, r):
        pltpu.make_async_copy(tab.at[srcs.at[b]], rows_bufs[r],
                              semrs[r]).wait()

    def block(i, bi):
        wait_idx(i + 1, (bi + 1) % 4)
        gather_rows((bi + 1) % 4, (bi + 1) % 2)
        wait_rows(bi, bi % 2)
        expand_scale(rows_bufs[bi % 2], ws.at[bi])
        pltpu.sync_copy(rowsf, acc.at[dsts.at[bi]], add=True)
        load_idx(i + 3, (bi + 3) % 4)

    load_idx(0, 0)
    load_idx(1, 1)
    load_idx(2, 2)
    wait_idx(0, 0)
    gather_rows(0, 0)

    def quad_body(p, carry):
        i = p * 4
        block(i, 0)
        block(i + 1, 1)
        block(i + 2, 2)
        block(i + 3, 3)
        return carry

    lax.fori_loop(0, (CPT2 - 3) // 4, quad_body, 0)
    e = CPT2 - 3
    for q in range(3):
        i = e + q
        bi = i % 4
        if q < 2:
            wait_idx(i + 1, (bi + 1) % 4)
            gather_rows((bi + 1) % 4, (bi + 1) % 2)
        wait_rows(bi, bi % 2)
        expand_scale(rows_bufs[bi % 2], ws.at[bi])
        pltpu.sync_copy(rowsf, acc.at[dsts.at[bi]], add=True)

    plsc.subcore_barrier()
    pltpu.sync_copy(acc.at[pl.ds(r0, ROWS_PER_TILE)],
                    out_hbm.at[c, pl.ds(r0, ROWS_PER_TILE)])


G = 2 * B
G_PER_TILE = G // NW
G_CHUNKS = G_PER_TILE // K


@functools.partial(
    pl.kernel,
    mesh=_mesh,
    out_type=jax.ShapeDtypeStruct((G, D), jnp.float32),
    scratch_types=[
        pltpu.VMEM((K,), jnp.int32),
        pltpu.VMEM((K, D), jnp.float32),
        pltpu.SemaphoreType.DMA,
    ],
)
def _sc_gather(tab_hbm, idx_hbm, out_hbm, idxv, rows, sem):
    c = lax.axis_index("c")
    s = lax.axis_index("s")
    wid = c * NS + s

    def body(i, carry):
        base = pl.multiple_of(wid * G_PER_TILE + i * K, 8)
        pltpu.sync_copy(idx_hbm.at[pl.ds(base, K)], idxv)
        pltpu.async_copy(tab_hbm.at[idxv], rows, sem).wait()
        pltpu.sync_copy(rows, out_hbm.at[pl.ds(base, K)])
        return carry

    lax.fori_loop(0, G_CHUNKS, body, 0)


_CB = 2000


def _pack_cols(v):
    a = jnp.concatenate([v[:, 32 * j:32 * j + 16] for j in range(D // 32)],
                        axis=1)
    b = jnp.concatenate(
        [v[:, 32 * j + 16:32 * j + 32] for j in range(D // 32)], axis=1)
    au = lax.bitcast_convert_type(a.astype(jnp.bfloat16),
                                  jnp.uint16).astype(jnp.uint32)
    bu = lax.bitcast_convert_type(b.astype(jnp.bfloat16),
                                  jnp.uint16).astype(jnp.uint32)
    return lax.bitcast_convert_type(au | (bu << 16), jnp.int32)


def _combine_body(p0_ref, p1_ref, h_ref, o_ref, o32_ref):
    v = DECAY * (p0_ref[...] + p1_ref[...]) + h_ref[...]
    o_ref[...] = v
    o32_ref[...] = _pack_cols(v)


def _combine(p, h):
    spec = pl.BlockSpec((_CB, D), lambda i: (i, 0))
    return pl.pallas_call(
        _combine_body,
        out_shape=[
            jax.ShapeDtypeStruct((N, D), jnp.float32),
            jax.ShapeDtypeStruct((N, D // 2), jnp.int32),
        ],
        grid=(N // _CB,),
        in_specs=[spec, spec, spec],
        out_specs=[spec, pl.BlockSpec((_CB, D // 2), lambda i: (i, 0))],
    )(p[0], p[1], h)


_MB = 1024


def _mlp_body(xs_ref, xd_ref, w1a_ref, w1b_ref, b1_ref, w2_ref, b2_ref,
              lab_ref, o_ref):
    i = pl.program_id(0)
    h = jnp.dot(xs_ref[...], w1a_ref[...], preferred_element_type=jnp.float32)
    h = h + jnp.dot(xd_ref[...], w1b_ref[...],
                    preferred_element_type=jnp.float32)
    h = jnp.maximum(h + b1_ref[...], 0.0)
    l0 = jnp.sum(h * w2_ref[0:1, :], axis=1) + b2_ref[0, 0]
    l1 = jnp.sum(h * w2_ref[1:2, :], axis=1) + b2_ref[0, 1]
    m = jnp.maximum(l0, l1)
    e0 = jnp.exp(l0 - m)
    e1 = jnp.exp(l1 - m)
    inv = 1.0 / (e0 + e1)
    p0 = e0 * inv
    p1 = e1 * inv
    m2 = jnp.maximum(p0, p1)
    ls = m2 + jnp.log(jnp.exp(p0 - m2) + jnp.exp(p1 - m2))
    lab = lab_ref[...][:, 0]
    lp = jnp.where(lab == 0, p0, p1) - ls
    part = (-jnp.sum(lp) * (1.0 / B)).reshape(1, 1)

    @pl.when(i == 0)
    def _init():
        o_ref[...] = jnp.zeros((1, 1), jnp.float32)

    o_ref[...] += part


def _mlp_loss(xsrc, xdst, w1a, w1b, b1, w2t, b2, labels):
    bspec = pl.BlockSpec((_MB, D), lambda i: (i, 0))
    wspec = pl.BlockSpec((D, D), lambda i: (0, 0))
    return pl.pallas_call(
        _mlp_body,
        out_shape=jax.ShapeDtypeStruct((1, 1), jnp.float32),
        grid=(B // _MB,),
        in_specs=[
            bspec, bspec, wspec, wspec,
            pl.BlockSpec((1, D), lambda i: (0, 0)),
            pl.BlockSpec((2, D), lambda i: (0, 0)),
            pl.BlockSpec((1, 2), lambda i: (0, 0)),
            pl.BlockSpec((_MB, 1), lambda i: (i, 0)),
        ],
        out_specs=pl.BlockSpec((1, 1), lambda i: (0, 0)),
    )(xsrc, xdst, w1a, w1b, b1, w2t, b2, labels)


def kernel(pairs, labels, edge_index, edge_weight, embedding, W1, b1, W2, b2):
    src = edge_index[0].astype(jnp.int32)
    dst = edge_index[1].astype(jnp.int32)
    w = edge_weight.astype(jnp.float32)

    pad = E_PAD2 - E
    src = jnp.concatenate([src, jnp.zeros((pad,), jnp.int32)])
    dst = jnp.concatenate([dst, jnp.zeros((pad,), jnp.int32)])
    w = jnp.concatenate([w, jnp.zeros((pad,), jnp.float32)])
    zeros = jnp.zeros((N_PAD, D), jnp.float32)
    tab_pad = jnp.zeros((N_PAD - N, D // 2), jnp.int32)

    h = embedding
    h32 = _pack_cols(embedding)
    for _ in range(LAYERS):
        p = _sc_layer(jnp.concatenate([h32, tab_pad]), src, dst, w, zeros)
        h, h32 = _combine(p[:, :N], h)

    idx_all = jnp.concatenate(
        [pairs[:, 0].astype(jnp.int32), pairs[:, 1].astype(jnp.int32)])
    x = _sc_gather(h, idx_all)
    xsrc = x[:B]
    xdst = x[B:]

    w1a = W1[:D]
    w1b = W1[D:]
    b1r = b1.reshape(1, D)
    w2t = W2.T
    b2r = b2.reshape(1, 2)
    lab = labels.astype(jnp.int32).reshape(B, 1)

    loss = _mlp_loss(xsrc, xdst, w1a, w1b, b1r, w2t, b2r, lab)
    return loss[0, 0]


# speedup vs baseline: 1.0329x; 1.0005x over previous
import functools
import math

import jax
import jax.numpy as jnp
from jax import lax
from jax.experimental import pallas as pl
from jax.experimental.pallas import tpu as pltpu
from jax.experimental.pallas import tpu_sc as plsc

N = 10000
D = 128
E = 320000
B = 16384
LAYERS = 2
DECAY = math.exp(-1.0)

NC = 2
NS = 16
NW = NC * NS
K = 128
CPT = -(-E // (NW * K))
E_PAD = CPT * NW * K
KC = 32
CPT2 = 315
E_PAD2 = CPT2 * NW * KC
N_PAD = 10112
ROWS_PER_TILE = N_PAD // NS

_mesh = plsc.VectorSubcoreMesh(core_axis_name="c", subcore_axis_name="s")


@functools.partial(
    pl.kernel,
    mesh=_mesh,
    out_type=jax.ShapeDtypeStruct((NC, N_PAD, D), jnp.float32),
    compiler_params=pltpu.CompilerParams(use_tc_tiling_on_sc=False),
    scratch_types=[
        pltpu.VMEM_SHARED((N_PAD, D), jnp.float32),
        pltpu.VMEM_SHARED((N_PAD, D // 2), jnp.int32),
        pltpu.VMEM((4, KC), jnp.int32),
        pltpu.VMEM((4, KC), jnp.int32),
        pltpu.VMEM((4, KC), jnp.float32),
        pltpu.VMEM((KC, D // 2), jnp.int32),
        pltpu.VMEM((KC, D // 2), jnp.int32),
        pltpu.VMEM((KC, D), jnp.float32),
        pltpu.SemaphoreType.DMA,
        pltpu.SemaphoreType.DMA,
        pltpu.SemaphoreType.DMA,
        pltpu.SemaphoreType.DMA,
        pltpu.SemaphoreType.DMA,
        pltpu.SemaphoreType.DMA,
    ],
)
def _sc_layer(h_hbm, src_hbm, dst_hbm, w_hbm, zero_hbm, out_hbm,
              acc, tab, srcs, dsts, ws, rows0, rows1, rowsf,
              semi0, semi1, semi2, semi3, semr0, semr1):
    c = lax.axis_index("c")
    s = lax.axis_index("s")
    wid = c * NS + s

    r0 = pl.multiple_of(s * ROWS_PER_TILE, 8)
    pltpu.sync_copy(zero_hbm.at[pl.ds(r0, ROWS_PER_TILE)],
                    acc.at[pl.ds(r0, ROWS_PER_TILE)])
    pltpu.sync_copy(h_hbm.at[pl.ds(r0, ROWS_PER_TILE)],
                    tab.at[pl.ds(r0, ROWS_PER_TILE)])
    plsc.subcore_barrier()

    semis = (semi0, semi1, semi2, semi3)
    rows_bufs = (rows0, rows1)
    semrs = (semr0, semr1)

    def load_idx(i, b):
        base = pl.multiple_of((wid * CPT2 + i) * KC, 8)
        pltpu.async_copy(src_hbm.at[pl.ds(base, KC)], srcs.at[b], semis[b])
        pltpu.async_copy(dst_hbm.at[pl.ds(base, KC)], dsts.at[b], semis[b])
        pltpu.async_copy(w_hbm.at[pl.ds(base, KC)], ws.at[b], semis[b])

    def wait_idx(i, b):
        base = pl.multiple_of((wid * CPT2 + i) * KC, 8)
        pltpu.make_async_copy(src_hbm.at[pl.ds(base, KC)], srcs.at[b],
                              semis[b]).wait()
        pltpu.make_async_copy(dst_hbm.at[pl.ds(base, KC)], dsts.at[b],
                              semis[b]).wait()
        pltpu.make_async_copy(w_hbm.at[pl.ds(base, KC)], ws.at[b],
                              semis[b]).wait()

    mask_hi = jnp.int32(-65536)

    def expand_scale(rows, wv):
        for g in range(KC // 16):
            w16 = wv[pl.ds(g * 16, 16)]
            for l in range(16):
                e = g * 16 + l
                web = jnp.full((16,), w16[l], jnp.float32)
                for j in range(D // 32):
                    x = rows[e, pl.ds(j * 16, 16)]
                    lo = lax.bitcast_convert_type(x << 16, jnp.float32)
                    hi = lax.bitcast_convert_type(x & mask_hi, jnp.float32)
                    rowsf[e, pl.ds(j * 32, 16)] = lo * web
                    rowsf[e, pl.ds(j * 32 + 16, 16)] = hi * web

    def gather_rows(b, r):
        pltpu.async_copy(tab.at[srcs.at[b]], rows_bufs[r], semrs[r])

    def wait_rows(b, r):
        pltpu.make_async_copy(tab.at[srcs.at[b]], rows_bufs[r],
                              semrs[r]).wait()

    def block(i, bi):
        wait_idx(i + 1, (bi + 1) % 4)
        gather_rows((bi + 1) % 4, (bi + 1) % 2)
        wait_rows(bi, bi % 2)
        expand_scale(rows_bufs[bi % 2], ws.at[bi])
        pltpu.sync_copy(rowsf, acc.at[dsts.at[bi]], add=True)
        load_idx(i + 3, (bi + 3) % 4)

    load_idx(0, 0)
    load_idx(1, 1)
    load_idx(2, 2)
    wait_idx(0, 0)
    gather_rows(0, 0)

    def quad_body(p, carry):
        i = p * 4
        block(i, 0)
        block(i + 1, 1)
        block(i + 2, 2)
        block(i + 3, 3)
        return carry

    lax.fori_loop(0, (CPT2 - 3) // 4, quad_body, 0)
    e = CPT2 - 3
    for q in range(3):
        i = e + q
        bi = i % 4
        if q < 2:
            wait_idx(i + 1, (bi + 1) % 4)
            gather_rows((bi + 1) % 4, (bi + 1) % 2)
        wait_rows(bi, bi % 2)
        expand_scale(rows_bufs[bi % 2], ws.at[bi])
        pltpu.sync_copy(rowsf, acc.at[dsts.at[bi]], add=True)

    plsc.subcore_barrier()
    pltpu.sync_copy(acc.at[pl.ds(r0, ROWS_PER_TILE)],
                    out_hbm.at[c, pl.ds(r0, ROWS_PER_TILE)])


G = 2 * B
G_PER_TILE = G // NW
G_CHUNKS = G_PER_TILE // K


@functools.partial(
    pl.kernel,
    mesh=_mesh,
    out_type=jax.ShapeDtypeStruct((G, D), jnp.float32),
    scratch_types=[
        pltpu.VMEM((K,), jnp.int32),
        pltpu.VMEM((K, D), jnp.float32),
        pltpu.SemaphoreType.DMA,
    ],
)
def _sc_gather(tab_hbm, idx_hbm, out_hbm, idxv, rows, sem):
    c = lax.axis_index("c")
    s = lax.axis_index("s")
    wid = c * NS + s

    def body(i, carry):
        base = pl.multiple_of(wid * G_PER_TILE + i * K, 8)
        pltpu.sync_copy(idx_hbm.at[pl.ds(base, K)], idxv)
        pltpu.async_copy(tab_hbm.at[idxv], rows, sem).wait()
        pltpu.sync_copy(rows, out_hbm.at[pl.ds(base, K)])
        return carry

    lax.fori_loop(0, G_CHUNKS, body, 0)


_CB = 2000


def _pack_cols(v):
    a = jnp.concatenate([v[:, 32 * j:32 * j + 16] for j in range(D // 32)],
                        axis=1)
    b = jnp.concatenate(
        [v[:, 32 * j + 16:32 * j + 32] for j in range(D // 32)], axis=1)
    au = lax.bitcast_convert_type(a.astype(jnp.bfloat16),
                                  jnp.uint16).astype(jnp.uint32)
    bu = lax.bitcast_convert_type(b.astype(jnp.bfloat16),
                                  jnp.uint16).astype(jnp.uint32)
    return lax.bitcast_convert_type(au | (bu << 16), jnp.int32)


def _combine_body(p0_ref, p1_ref, h_ref, o_ref, o32_ref):
    v = DECAY * (p0_ref[...] + p1_ref[...]) + h_ref[...]
    o_ref[...] = v
    o32_ref[...] = _pack_cols(v)


def _combine(p, h):
    spec = pl.BlockSpec((_CB, D), lambda i: (i, 0))
    return pl.pallas_call(
        _combine_body,
        out_shape=[
            jax.ShapeDtypeStruct((N, D), jnp.float32),
            jax.ShapeDtypeStruct((N, D // 2), jnp.int32),
        ],
        grid=(N // _CB,),
        in_specs=[spec, spec, spec],
        out_specs=[spec, pl.BlockSpec((_CB, D // 2), lambda i: (i, 0))],
    )(p[0], p[1], h)


_MB = 1024


def _mlp_body(xs_ref, xd_ref, w1a_ref, w1b_ref, b1_ref, w2_ref, b2_ref,
              lab_ref, o_ref):
    i = pl.program_id(0)
    h = jnp.dot(xs_ref[...].astype(jnp.bfloat16),
                w1a_ref[...].astype(jnp.bfloat16),
                preferred_element_type=jnp.float32)
    h = h + jnp.dot(xd_ref[...].astype(jnp.bfloat16),
                    w1b_ref[...].astype(jnp.bfloat16),
                    preferred_element_type=jnp.float32)
    h = jnp.maximum(h + b1_ref[...], 0.0)
    l0 = jnp.sum(h * w2_ref[0:1, :], axis=1) + b2_ref[0, 0]
    l1 = jnp.sum(h * w2_ref[1:2, :], axis=1) + b2_ref[0, 1]
    m = jnp.maximum(l0, l1)
    e0 = jnp.exp(l0 - m)
    e1 = jnp.exp(l1 - m)
    inv = 1.0 / (e0 + e1)
    p0 = e0 * inv
    p1 = e1 * inv
    m2 = jnp.maximum(p0, p1)
    ls = m2 + jnp.log(jnp.exp(p0 - m2) + jnp.exp(p1 - m2))
    lab = lab_ref[...][:, 0]
    lp = jnp.where(lab == 0, p0, p1) - ls
    part = (-jnp.sum(lp) * (1.0 / B)).reshape(1, 1)

    @pl.when(i == 0)
    def _init():
        o_ref[...] = jnp.zeros((1, 1), jnp.float32)

    o_ref[...] += part


def _mlp_loss(xsrc, xdst, w1a, w1b, b1, w2t, b2, labels):
    bspec = pl.BlockSpec((_MB, D), lambda i: (i, 0))
    wspec = pl.BlockSpec((D, D), lambda i: (0, 0))
    return pl.pallas_call(
        _mlp_body,
        out_shape=jax.ShapeDtypeStruct((1, 1), jnp.float32),
        grid=(B // _MB,),
        in_specs=[
            bspec, bspec, wspec, wspec,
            pl.BlockSpec((1, D), lambda i: (0, 0)),
            pl.BlockSpec((2, D), lambda i: (0, 0)),
            pl.BlockSpec((1, 2), lambda i: (0, 0)),
            pl.BlockSpec((_MB, 1), lambda i: (i, 0)),
        ],
        out_specs=pl.BlockSpec((1, 1), lambda i: (0, 0)),
    )(xsrc, xdst, w1a, w1b, b1, w2t, b2, labels)


def kernel(pairs, labels, edge_index, edge_weight, embedding, W1, b1, W2, b2):
    src = edge_index[0].astype(jnp.int32)
    dst = edge_index[1].astype(jnp.int32)
    w = edge_weight.astype(jnp.float32)

    pad = E_PAD2 - E
    src = jnp.concatenate([src, jnp.zeros((pad,), jnp.int32)])
    dst = jnp.concatenate([dst, jnp.zeros((pad,), jnp.int32)])
    w = jnp.concatenate([w, jnp.zeros((pad,), jnp.float32)])
    zeros = jnp.zeros((N_PAD, D), jnp.float32)
    tab_pad = jnp.zeros((N_PAD - N, D // 2), jnp.int32)

    h = embedding
    h32 = _pack_cols(embedding)
    for _ in range(LAYERS):
        p = _sc_layer(jnp.concatenate([h32, tab_pad]), src, dst, w, zeros)
        h, h32 = _combine(p[:, :N], h)

    idx_all = jnp.concatenate(
        [pairs[:, 0].astype(jnp.int32), pairs[:, 1].astype(jnp.int32)])
    x = _sc_gather(h, idx_all)
    xsrc = x[:B]
    xdst = x[B:]

    w1a = W1[:D]
    w1b = W1[D:]
    b1r = b1.reshape(1, D)
    w2t = W2.T
    b2r = b2.reshape(1, 2)
    lab = labels.astype(jnp.int32).reshape(B, 1)

    loss = _mlp_loss(xsrc, xdst, w1a, w1b, b1r, w2t, b2r, lab)
    return loss[0, 0]


# speedup vs baseline: 1.0556x; 1.0220x over previous
import functools
import math

import jax
import jax.numpy as jnp
from jax import lax
from jax.experimental import pallas as pl
from jax.experimental.pallas import tpu as pltpu
from jax.experimental.pallas import tpu_sc as plsc

N = 10000
D = 128
E = 320000
B = 16384
LAYERS = 2
DECAY = math.exp(-1.0)

NC = 2
NS = 16
NW = NC * NS
K = 128
CPT = -(-E // (NW * K))
E_PAD = CPT * NW * K
KC = 32
CPT2 = 315
E_PAD2 = CPT2 * NW * KC
N_PAD = 10112
ROWS_PER_TILE = N_PAD // NS

_mesh = plsc.VectorSubcoreMesh(core_axis_name="c", subcore_axis_name="s")


@functools.partial(
    pl.kernel,
    mesh=_mesh,
    out_type=jax.ShapeDtypeStruct((NC, N_PAD, D), jnp.float32),
    compiler_params=pltpu.CompilerParams(use_tc_tiling_on_sc=False),
    scratch_types=[
        pltpu.VMEM_SHARED((N_PAD, D), jnp.float32),
        pltpu.VMEM_SHARED((N_PAD, D // 2), jnp.int32),
        pltpu.VMEM((4, KC), jnp.int32),
        pltpu.VMEM((4, KC), jnp.int32),
        pltpu.VMEM((4, KC), jnp.float32),
        pltpu.VMEM((KC, D // 2), jnp.int32),
        pltpu.VMEM((KC, D // 2), jnp.int32),
        pltpu.VMEM((KC, D), jnp.float32),
        pltpu.SemaphoreType.DMA,
        pltpu.SemaphoreType.DMA,
        pltpu.SemaphoreType.DMA,
        pltpu.SemaphoreType.DMA,
        pltpu.SemaphoreType.DMA,
        pltpu.SemaphoreType.DMA,
    ],
)
def _sc_layer(h_hbm, src_hbm, dst_hbm, w_hbm, zero_hbm, out_hbm,
              acc, tab, srcs, dsts, ws, rows0, rows1, rowsf,
              semi0, semi1, semi2, semi3, semr0, semr1):
    c = lax.axis_index("c")
    s = lax.axis_index("s")
    wid = c * NS + s

    r0 = pl.multiple_of(s * ROWS_PER_TILE, 8)
    pltpu.sync_copy(zero_hbm.at[pl.ds(r0, ROWS_PER_TILE)],
                    acc.at[pl.ds(r0, ROWS_PER_TILE)])
    pltpu.sync_copy(h_hbm.at[pl.ds(r0, ROWS_PER_TILE)],
                    tab.at[pl.ds(r0, ROWS_PER_TILE)])
    plsc.subcore_barrier()

    semis = (semi0, semi1, semi2, semi3)
    rows_bufs = (rows0, rows1)
    semrs = (semr0, semr1)

    def load_idx(i, b):
        base = pl.multiple_of((wid * CPT2 + i) * KC, 8)
        pltpu.async_copy(src_hbm.at[pl.ds(base, KC)], srcs.at[b], semis[b])
        pltpu.async_copy(dst_hbm.at[pl.ds(base, KC)], dsts.at[b], semis[b])
        pltpu.async_copy(w_hbm.at[pl.ds(base, KC)], ws.at[b], semis[b])

    def wait_idx(i, b):
        base = pl.multiple_of((wid * CPT2 + i) * KC, 8)
        pltpu.make_async_copy(src_hbm.at[pl.ds(base, KC)], srcs.at[b],
                              semis[b]).wait()
        pltpu.make_async_copy(dst_hbm.at[pl.ds(base, KC)], dsts.at[b],
                              semis[b]).wait()
        pltpu.make_async_copy(w_hbm.at[pl.ds(base, KC)], ws.at[b],
                              semis[b]).wait()

    mask_hi = jnp.int32(-65536)

    def expand_scale(rows, wv):
        for g in range(KC // 16):
            w16 = wv[pl.ds(g * 16, 16)]
            for l in range(16):
                e = g * 16 + l
                web = jnp.full((16,), w16[l], jnp.float32)
                for j in range(D // 32):
                    x = rows[e, pl.ds(j * 16, 16)]
                    lo = lax.bitcast_convert_type(x << 16, jnp.float32)
                    hi = lax.bitcast_convert_type(x & mask_hi, jnp.float32)
                    rowsf[e, pl.ds(j * 32, 16)] = lo * web
                    rowsf[e, pl.ds(j * 32 + 16, 16)] = hi * web

    def gather_rows(b, r):
        pltpu.async_copy(tab.at[srcs.at[b]], rows_bufs[r], semrs[r])

    def wait_rows(b, r):
        pltpu.make_async_copy(tab.at[srcs.at[b]], rows_bufs[r],
                              semrs[r]).wait()

    def block(i, bi):
        wait_idx(i + 1, (bi + 1) % 4)
        gather_rows((bi + 1) % 4, (bi + 1) % 2)
        wait_rows(bi, bi % 2)
        expand_scale(rows_bufs[bi % 2], ws.at[bi])
        pltpu.sync_copy(rowsf, acc.at[dsts.at[bi]], add=True)
        load_idx(i + 3, (bi + 3) % 4)

    load_idx(0, 0)
    load_idx(1, 1)
    load_idx(2, 2)
    wait_idx(0, 0)
    gather_rows(0, 0)

    def quad_body(p, carry):
        i = p * 4
        block(i, 0)
        block(i + 1, 1)
        block(i + 2, 2)
        block(i + 3, 3)
        return carry

    lax.fori_loop(0, (CPT2 - 3) // 4, quad_body, 0)
    e = CPT2 - 3
    for q in range(3):
        i = e + q
        bi = i % 4
        if q < 2:
            wait_idx(i + 1, (bi + 1) % 4)
            gather_rows((bi + 1) % 4, (bi + 1) % 2)
        wait_rows(bi, bi % 2)
        expand_scale(rows_bufs[bi % 2], ws.at[bi])
        pltpu.sync_copy(rowsf, acc.at[dsts.at[bi]], add=True)

    plsc.subcore_barrier()
    pltpu.sync_copy(acc.at[pl.ds(r0, ROWS_PER_TILE)],
                    out_hbm.at[c, pl.ds(r0, ROWS_PER_TILE)])


G = 2 * B
G_PER_TILE = G // NW
G_CHUNKS = G_PER_TILE // K


@functools.partial(
    pl.kernel,
    mesh=_mesh,
    out_type=[
        jax.ShapeDtypeStruct((B, D), jnp.float32),
        jax.ShapeDtypeStruct((B, D), jnp.float32),
    ],
    scratch_types=[
        pltpu.VMEM((K,), jnp.int32),
        pltpu.VMEM((K, D), jnp.float32),
        pltpu.SemaphoreType.DMA,
    ],
)
def _sc_gather(tab_hbm, idx_hbm, out1_hbm, out2_hbm, idxv, rows, sem):
    c = lax.axis_index("c")
    s = lax.axis_index("s")

    def body(i, carry):
        local = pl.multiple_of(s * (B // NS) + i * K, 8)
        base = pl.multiple_of(c * B + local, 8)
        pltpu.sync_copy(idx_hbm.at[pl.ds(base, K)], idxv)
        pltpu.async_copy(tab_hbm.at[idxv], rows, sem).wait()

        @pl.when(c == 0)
        def _():
            pltpu.sync_copy(rows, out1_hbm.at[pl.ds(local, K)])

        @pl.when(c == 1)
        def _():
            pltpu.sync_copy(rows, out2_hbm.at[pl.ds(local, K)])

        return carry

    lax.fori_loop(0, (B // NS) // K, body, 0)


_CB = 2000


def _pack_cols(v):
    a = jnp.concatenate([v[:, 32 * j:32 * j + 16] for j in range(D // 32)],
                        axis=1)
    b = jnp.concatenate(
        [v[:, 32 * j + 16:32 * j + 32] for j in range(D // 32)], axis=1)
    au = lax.bitcast_convert_type(a.astype(jnp.bfloat16),
                                  jnp.uint16).astype(jnp.uint32)
    bu = lax.bitcast_convert_type(b.astype(jnp.bfloat16),
                                  jnp.uint16).astype(jnp.uint32)
    return lax.bitcast_convert_type(au | (bu << 16), jnp.int32)


def _combine_body(p0_ref, p1_ref, h_ref, o_ref, o32_ref):
    v = DECAY * (p0_ref[...] + p1_ref[...]) + h_ref[...]
    o_ref[...] = v
    o32_ref[...] = _pack_cols(v)


def _combine(p, h):
    spec = pl.BlockSpec((_CB, D), lambda i: (i, 0))
    return pl.pallas_call(
        _combine_body,
        out_shape=[
            jax.ShapeDtypeStruct((N, D), jnp.float32),
            jax.ShapeDtypeStruct((N, D // 2), jnp.int32),
        ],
        grid=(N // _CB,),
        in_specs=[spec, spec, spec],
        out_specs=[spec, pl.BlockSpec((_CB, D // 2), lambda i: (i, 0))],
    )(p[0], p[1], h)


_MB = 1024


def _mlp_body(xs_ref, xd_ref, w1a_ref, w1b_ref, b1_ref, w2_ref, b2_ref,
              lab_ref, o_ref):
    i = pl.program_id(0)
    h = jnp.dot(xs_ref[...].astype(jnp.bfloat16),
                w1a_ref[...].astype(jnp.bfloat16),
                preferred_element_type=jnp.float32)
    h = h + jnp.dot(xd_ref[...].astype(jnp.bfloat16),
                    w1b_ref[...].astype(jnp.bfloat16),
                    preferred_element_type=jnp.float32)
    h = jnp.maximum(h + b1_ref[...], 0.0)
    l0 = jnp.sum(h * w2_ref[0:1, :], axis=1) + b2_ref[0, 0]
    l1 = jnp.sum(h * w2_ref[1:2, :], axis=1) + b2_ref[0, 1]
    m = jnp.maximum(l0, l1)
    e0 = jnp.exp(l0 - m)
    e1 = jnp.exp(l1 - m)
    inv = 1.0 / (e0 + e1)
    p0 = e0 * inv
    p1 = e1 * inv
    m2 = jnp.maximum(p0, p1)
    ls = m2 + jnp.log(jnp.exp(p0 - m2) + jnp.exp(p1 - m2))
    lab = lab_ref[...][:, 0]
    lp = jnp.where(lab == 0, p0, p1) - ls
    part = (-jnp.sum(lp) * (1.0 / B)).reshape(1, 1)

    @pl.when(i == 0)
    def _init():
        o_ref[...] = jnp.zeros((1, 1), jnp.float32)

    o_ref[...] += part


def _mlp_loss(xsrc, xdst, w1a, w1b, b1, w2t, b2, labels):
    bspec = pl.BlockSpec((_MB, D), lambda i: (i, 0))
    wspec = pl.BlockSpec((D, D), lambda i: (0, 0))
    return pl.pallas_call(
        _mlp_body,
        out_shape=jax.ShapeDtypeStruct((1, 1), jnp.float32),
        grid=(B // _MB,),
        in_specs=[
            bspec, bspec, wspec, wspec,
            pl.BlockSpec((1, D), lambda i: (0, 0)),
            pl.BlockSpec((2, D), lambda i: (0, 0)),
            pl.BlockSpec((1, 2), lambda i: (0, 0)),
            pl.BlockSpec((_MB, 1), lambda i: (i, 0)),
        ],
        out_specs=pl.BlockSpec((1, 1), lambda i: (0, 0)),
    )(xsrc, xdst, w1a, w1b, b1, w2t, b2, labels)


def kernel(pairs, labels, edge_index, edge_weight, embedding, W1, b1, W2, b2):
    src = edge_index[0].astype(jnp.int32)
    dst = edge_index[1].astype(jnp.int32)
    w = edge_weight.astype(jnp.float32)

    pad = E_PAD2 - E
    src = jnp.concatenate([src, jnp.zeros((pad,), jnp.int32)])
    dst = jnp.concatenate([dst, jnp.zeros((pad,), jnp.int32)])
    w = jnp.concatenate([w, jnp.zeros((pad,), jnp.float32)])
    zeros = jnp.zeros((N_PAD, D), jnp.float32)
    tab_pad = jnp.zeros((N_PAD - N, D // 2), jnp.int32)

    h = embedding
    h32 = _pack_cols(embedding)
    for _ in range(LAYERS):
        p = _sc_layer(jnp.concatenate([h32, tab_pad]), src, dst, w, zeros)
        h, h32 = _combine(p[:, :N], h)

    idx_all = jnp.concatenate(
        [pairs[:, 0].astype(jnp.int32), pairs[:, 1].astype(jnp.int32)])
    xsrc, xdst = _sc_gather(h, idx_all)

    w1a = W1[:D]
    w1b = W1[D:]
    b1r = b1.reshape(1, D)
    w2t = W2.T
    b2r = b2.reshape(1, 2)
    lab = labels.astype(jnp.int32).reshape(B, 1)

    loss = _mlp_loss(xsrc, xdst, w1a, w1b, b1r, w2t, b2r, lab)
    return loss[0, 0]
